# baseline (speedup 1.0000x reference)
import jax
import jax.numpy as jnp
from jax import lax
from jax.experimental import pallas as pl
from jax.experimental.pallas import tpu as pltpu
from jax.experimental.pallas import tpu_sc as plsc

_N, _P, _G, _C = 8, 8732, 64, 21
_P_TILE = 1792
_INT_MIN = -(2 ** 31)


def _stage1_body(pred_ref, gtx_ref, dfx_ref, lconf_ref, lloc_ref, many_ref,
                 mf_ref):
    predT = pred_ref[0].T
    logits = predT[4:25, :]
    m = jnp.max(logits, axis=0, keepdims=True)
    lse = jnp.log(jnp.sum(jnp.exp(logits - m), axis=0, keepdims=True)) + m
    logp = logits - lse

    gtx = gtx_ref[0]
    gclsT = gtx[:, 9:30].T

    d_l = dfx_ref[0:1, :]
    d_r = dfx_ref[1:2, :]
    d_b = dfx_ref[2:3, :]
    d_t = dfx_ref[3:4, :]
    d_area = dfx_ref[4:5, :]
    inv_dw = dfx_ref[5:6, :]
    inv_dh = dfx_ref[6:7, :]

    a_cx = predT[0:1, :] + dfx_ref[7:8, :]
    a_cy = predT[1:2, :] + dfx_ref[8:9, :]
    a_w = predT[2:3, :] + dfx_ref[9:10, :]
    a_h = predT[3:4, :] + dfx_ref[10:11, :]

    def sl1(x):
        ax = jnp.abs(x)
        mm = jnp.minimum(ax, 1.0)
        return (ax - mm) + (0.5 * mm) * mm

    acc = jnp.zeros((8, _P_TILE), jnp.float32)
    for c in range(_G // 8):
        sl = slice(c * 8, c * 8 + 8)
        g_cx = gtx[sl, 0:1]
        g_cy = gtx[sl, 1:2]
        g_l = gtx[sl, 2:3]
        g_r = gtx[sl, 3:4]
        g_b = gtx[sl, 4:5]
        g_t = gtx[sl, 5:6]
        g_area = gtx[sl, 6:7]
        log_gw = gtx[sl, 7:8]
        log_gh = gtx[sl, 8:9]
        w = jnp.maximum(jnp.minimum(g_r, d_r) - jnp.maximum(g_l, d_l), 0.0)
        h = jnp.maximum(jnp.minimum(g_t, d_t) - jnp.maximum(g_b, d_b), 0.0)
        inter = w * h
        match = (3.0 * inter) > (g_area + d_area)
        mf_ref[sl, :] = match.astype(jnp.float32)
        x1 = a_cx - g_cx * inv_dw
        x2 = a_cy - g_cy * inv_dh
        x3 = a_w - log_gw
        x4 = a_h - log_gh
        s = sl1(x1) + sl1(x2) + sl1(x3) + sl1(x4)
        acc = acc + jnp.where(match, s, 0.0)

    lloc = jnp.sum(acc, axis=0, keepdims=True)
    mm_ = jnp.dot(gclsT, mf_ref[...], preferred_element_type=jnp.float32)
    cnt = jnp.sum(mm_, axis=0, keepdims=True)
    lcp = jnp.sum(logp * mm_, axis=0, keepdims=True)
    lc = jnp.where(cnt > 0.0, -lcp, logp[0:1, :])

    lconf_ref[0] = lc
    lloc_ref[0] = lloc
    many_ref[0] = (cnt > 0.0).astype(jnp.float32)


_P_PAD = 8736
_NV = _P_PAD // 16


def _sc_stage2_body(lconf_hbm, lloc_hbm, many_hbm, a_hbm, out_hbm,
                    lc_v, ll_v, my_v, uk_v, av_v, hist_v, res_v):
    imin = jnp.int32(_INT_MIN)
    wid = lax.axis_index("s") * 2 + lax.axis_index("c")

    @pl.when(wid < _N)
    def _work():
        lanes = lax.iota(jnp.int32, 16)
        pltpu.sync_copy(lconf_hbm.at[wid], lc_v)
        pltpu.sync_copy(lloc_hbm.at[wid], ll_v)
        pltpu.sync_copy(many_hbm.at[wid], my_v)
        pltpu.sync_copy(a_hbm, av_v)

        def build(v, _):
            f = lax.bitcast_convert_type(lc_v[pl.ds(v * 16, 16)], jnp.int32)
            u = jnp.where(f < 0, ~f, f ^ imin)
            u = jnp.where(v * 16 + lanes < _P, u, jnp.int32(-1))
            uk_v[pl.ds(v * 16, 16)] = u
            return 0
        lax.fori_loop(0, _NV, build, 0)

        def cntm(v, acc):
            mv = my_v[pl.ds(v * 16, 16)]
            return acc + jnp.where(v * 16 + lanes < _P, mv, 0.0)
        posf = jnp.sum(lax.fori_loop(0, _NV, cntm,
                                     jnp.zeros((16,), jnp.float32)))
        pos_orig = posf.astype(jnp.int32)
        pos = jnp.maximum(pos_orig, 1)
        neg = jnp.maximum(jnp.minimum(_P - pos_orig, 3 * pos), 1)

        ones_i = jnp.ones((16,), jnp.int32)

        def select(k0):
            def pass_body(p, carry):
                prefix, hmask, k = carry
                shift = 28 - p * 4
                for b in range(16):
                    hist_v[pl.ds(b * 16, 16)] = jnp.zeros((16,), jnp.int32)

                def scan(v, _):
                    u = uk_v[pl.ds(v * 16, 16)]
                    cand = (u & hmask) == prefix
                    shv = jnp.full((16,), shift, jnp.int32)
                    binv = lax.shift_right_logical(u, shv) & 15
                    idx = binv * 16 + lanes
                    plsc.addupdate_scatter(hist_v, [idx], ones_i, mask=cand)
                    return 0
                lax.fori_loop(0, _NV, scan, 0)

                bincnt = jnp.zeros((16,), jnp.int32)
                for b in range(16):
                    cb = jnp.sum(hist_v[pl.ds(b * 16, 16)])
                    bincnt = jnp.where(lanes == b, cb, bincnt)
                incl = plsc.cumsum(bincnt)
                excl = incl - bincnt
                bsel = plsc.all_reduce_ffs(incl >= k)
                bs = bsel if getattr(bsel, "ndim", 0) == 0 else jnp.max(bsel)
                below = jnp.sum(jnp.where(lanes == bs, excl, 0))
                prefix = prefix | lax.shift_left(bs, shift)
                hmask = hmask | lax.shift_left(jnp.int32(15), shift)
                return prefix, hmask, k - below

            pfx, _, _ = lax.fori_loop(
                0, 8, pass_body, (jnp.int32(0), jnp.int32(0), k0))
            return pfx

        slo = select(neg) ^ imin
        shi = select(_P - pos + 1) ^ imin

        av = av_v[...]

        def csum(v, acc):
            sk = uk_v[pl.ds(v * 16, 16)] ^ imin
            va = ((sk < slo) | (sk > shi)) & (v * 16 + lanes < _P)
            lcv = lc_v[pl.ds(v * 16, 16)]
            llv = ll_v[pl.ds(v * 16, 16)]
            return acc + jnp.where(va, llv + av * jnp.abs(lcv), 0.0)
        accc = lax.fori_loop(0, _NV, csum, jnp.zeros((16,), jnp.float32))
        row = jnp.sum(accc)
        posv = jnp.full((16,), pos.astype(jnp.float32))
        res_v[...] = jnp.where(lanes == 0, row, 0.0) / posv
        pltpu.sync_copy(res_v, out_hbm.at[wid])


def kernel(pred_bboxes, default_bboxes, gt_bboxes, a=1):
    dcx = default_bboxes[:, 0]
    dcy = default_bboxes[:, 1]
    dw = default_bboxes[:, 2]
    dh = default_bboxes[:, 3]
    dfx = jnp.stack([
        dcx - dw * 0.5, dcx + dw * 0.5, dcy - dh * 0.5, dcy + dh * 0.5,
        dw * dh, 1.0 / dw, 1.0 / dh, dcx / dw, dcy / dh,
        jnp.log(dw), jnp.log(dh)], axis=0)

    gcx = gt_bboxes[..., 0]
    gcy = gt_bboxes[..., 1]
    gw = gt_bboxes[..., 2]
    gh = gt_bboxes[..., 3]
    gtx = jnp.concatenate([
        jnp.stack([gcx, gcy, gcx - gw * 0.5, gcx + gw * 0.5,
                   gcy - gh * 0.5, gcy + gh * 0.5, gw * gh,
                   jnp.log(gw), jnp.log(gh)], axis=-1),
        gt_bboxes[..., 4:25],
        jnp.zeros((_N, _G, 2), jnp.float32)], axis=-1)

    n_tiles = (_P + _P_TILE - 1) // _P_TILE
    out2 = jax.ShapeDtypeStruct((_N, 1, _P_PAD), jnp.float32)
    lconf, lloc, many = pl.pallas_call(
        _stage1_body,
        grid=(_N, n_tiles),
        in_specs=[
            pl.BlockSpec((1, _P_TILE, 25), lambda n, t: (n, t, 0)),
            pl.BlockSpec((1, _G, 32), lambda n, t: (n, 0, 0)),
            pl.BlockSpec((11, _P_TILE), lambda n, t: (0, t)),
        ],
        out_specs=[
            pl.BlockSpec((1, 1, _P_TILE), lambda n, t: (n, 0, t)),
            pl.BlockSpec((1, 1, _P_TILE), lambda n, t: (n, 0, t)),
            pl.BlockSpec((1, 1, _P_TILE), lambda n, t: (n, 0, t)),
        ],
        out_shape=[out2, out2, out2],
        scratch_shapes=[pltpu.VMEM((_G, _P_TILE), jnp.float32)],
    )(pred_bboxes, gtx, dfx)

    a_arr = jnp.full((16,), a, jnp.float32)
    import functools
    sc_call = functools.partial(
        pl.kernel,
        out_type=jax.ShapeDtypeStruct((_N, 16), jnp.float32),
        mesh=plsc.VectorSubcoreMesh(core_axis_name="c", subcore_axis_name="s"),
        compiler_params=pltpu.CompilerParams(needs_layout_passes=False),
        scratch_types=[
            pltpu.VMEM((_P_PAD,), jnp.float32),
            pltpu.VMEM((_P_PAD,), jnp.float32),
            pltpu.VMEM((_P_PAD,), jnp.float32),
            pltpu.VMEM((_P_PAD,), jnp.int32),
            pltpu.VMEM((16,), jnp.float32),
            pltpu.VMEM((256,), jnp.int32),
            pltpu.VMEM((16,), jnp.float32),
        ],
    )(_sc_stage2_body)
    out = sc_call(lconf.reshape(_N, _P_PAD), lloc.reshape(_N, _P_PAD),
                  many.reshape(_N, _P_PAD), a_arr)
    return jnp.sum(out) / float(_N)


# speedup vs baseline: 1.0524x; 1.0524x over previous
import jax
import jax.numpy as jnp
from jax import lax
from jax.experimental import pallas as pl
from jax.experimental.pallas import tpu as pltpu
from jax.experimental.pallas import tpu_sc as plsc

_N, _P, _G, _C = 8, 8732, 64, 21
_P_TILE = 1792
_INT_MIN = -(2 ** 31)


def _stage1_body(pred_ref, gtx_ref, dfx_ref, lconf_ref, lloc_ref, many_ref,
                 mf_ref):
    predT = pred_ref[0].T
    logits = predT[4:25, :]
    m = jnp.max(logits, axis=0, keepdims=True)
    lse = jnp.log(jnp.sum(jnp.exp(logits - m), axis=0, keepdims=True)) + m
    logp = logits - lse

    gtx = gtx_ref[0]
    gclsT = gtx[:, 9:30].T

    d_l = dfx_ref[0:1, :]
    d_r = dfx_ref[1:2, :]
    d_b = dfx_ref[2:3, :]
    d_t = dfx_ref[3:4, :]
    d_area = dfx_ref[4:5, :]
    inv_dw = dfx_ref[5:6, :]
    inv_dh = dfx_ref[6:7, :]

    a_cx = predT[0:1, :] + dfx_ref[7:8, :]
    a_cy = predT[1:2, :] + dfx_ref[8:9, :]
    a_w = predT[2:3, :] + dfx_ref[9:10, :]
    a_h = predT[3:4, :] + dfx_ref[10:11, :]

    def sl1(x):
        ax = jnp.abs(x)
        mm = jnp.minimum(ax, 1.0)
        return (ax - mm) + (0.5 * mm) * mm

    acc = jnp.zeros((8, _P_TILE), jnp.float32)
    for c in range(_G // 8):
        sl = slice(c * 8, c * 8 + 8)
        g_cx = gtx[sl, 0:1]
        g_cy = gtx[sl, 1:2]
        g_l = gtx[sl, 2:3]
        g_r = gtx[sl, 3:4]
        g_b = gtx[sl, 4:5]
        g_t = gtx[sl, 5:6]
        g_area = gtx[sl, 6:7]
        log_gw = gtx[sl, 7:8]
        log_gh = gtx[sl, 8:9]
        w = jnp.maximum(jnp.minimum(g_r, d_r) - jnp.maximum(g_l, d_l), 0.0)
        h = jnp.maximum(jnp.minimum(g_t, d_t) - jnp.maximum(g_b, d_b), 0.0)
        inter = w * h
        match = (3.0 * inter) > (g_area + d_area)
        mf_ref[sl, :] = match.astype(jnp.float32)
        x1 = a_cx - g_cx * inv_dw
        x2 = a_cy - g_cy * inv_dh
        x3 = a_w - log_gw
        x4 = a_h - log_gh
        s = sl1(x1) + sl1(x2) + sl1(x3) + sl1(x4)
        acc = acc + jnp.where(match, s, 0.0)

    lloc = jnp.sum(acc, axis=0, keepdims=True)
    mm_ = jnp.dot(gclsT, mf_ref[...], preferred_element_type=jnp.float32)
    cnt = jnp.sum(mm_, axis=0, keepdims=True)
    lcp = jnp.sum(logp * mm_, axis=0, keepdims=True)
    lc = jnp.where(cnt > 0.0, -lcp, logp[0:1, :])

    lconf_ref[0] = lc
    lloc_ref[0] = lloc
    many_ref[0] = (cnt > 0.0).astype(jnp.float32)


_P_PAD = 8736
_NV = _P_PAD // 16


def _sc_stage2_body(lconf_hbm, lloc_hbm, many_hbm, a_hbm, out_hbm,
                    lc_v, ll_v, my_v, uk_v, av_v, hist_v, res_v):
    imin = jnp.int32(_INT_MIN)
    wid = lax.axis_index("s") * 2 + lax.axis_index("c")

    @pl.when(wid < _N)
    def _work():
        lanes = lax.iota(jnp.int32, 16)
        pltpu.sync_copy(lconf_hbm.at[wid], lc_v)
        pltpu.sync_copy(lloc_hbm.at[wid], ll_v)
        pltpu.sync_copy(many_hbm.at[wid], my_v)
        pltpu.sync_copy(a_hbm, av_v)

        def build(v, _):
            f = lax.bitcast_convert_type(lc_v[pl.ds(v * 16, 16)], jnp.int32)
            u = jnp.where(f < 0, ~f, f ^ imin)
            u = jnp.where(v * 16 + lanes < _P, u, jnp.int32(-1))
            uk_v[pl.ds(v * 16, 16)] = u
            return 0
        lax.fori_loop(0, _NV, build, 0, unroll=13)

        def cntm(v, acc):
            mv = my_v[pl.ds(v * 16, 16)]
            return acc + jnp.where(v * 16 + lanes < _P, mv, 0.0)
        posf = jnp.sum(lax.fori_loop(0, _NV, cntm, jnp.zeros((16,), jnp.float32), unroll=13))
        pos_orig = posf.astype(jnp.int32)
        pos = jnp.maximum(pos_orig, 1)
        neg = jnp.maximum(jnp.minimum(_P - pos_orig, 3 * pos), 1)

        ones_i = jnp.ones((16,), jnp.int32)

        def select(k0):
            def pass_body(p, carry):
                prefix, hmask, k = carry
                shift = 28 - p * 4
                for b in range(16):
                    hist_v[pl.ds(b * 16, 16)] = jnp.zeros((16,), jnp.int32)

                def scan(v, _):
                    u = uk_v[pl.ds(v * 16, 16)]
                    cand = (u & hmask) == prefix
                    shv = jnp.full((16,), shift, jnp.int32)
                    binv = lax.shift_right_logical(u, shv) & 15
                    idx = binv * 16 + lanes
                    plsc.addupdate_scatter(hist_v, [idx], ones_i, mask=cand)
                    return 0
                lax.fori_loop(0, _NV, scan, 0, unroll=13)

                bincnt = jnp.zeros((16,), jnp.int32)
                for b in range(16):
                    cb = jnp.sum(hist_v[pl.ds(b * 16, 16)])
                    bincnt = jnp.where(lanes == b, cb, bincnt)
                incl = plsc.cumsum(bincnt)
                excl = incl - bincnt
                bsel = plsc.all_reduce_ffs(incl >= k)
                bs = bsel if getattr(bsel, "ndim", 0) == 0 else jnp.max(bsel)
                below = jnp.sum(jnp.where(lanes == bs, excl, 0))
                prefix = prefix | lax.shift_left(bs, shift)
                hmask = hmask | lax.shift_left(jnp.int32(15), shift)
                return prefix, hmask, k - below

            pfx, _, _ = lax.fori_loop(
                0, 8, pass_body, (jnp.int32(0), jnp.int32(0), k0))
            return pfx

        slo = select(neg) ^ imin
        shi = select(_P - pos + 1) ^ imin

        av = av_v[...]

        def csum(v, acc):
            sk = uk_v[pl.ds(v * 16, 16)] ^ imin
            va = ((sk < slo) | (sk > shi)) & (v * 16 + lanes < _P)
            lcv = lc_v[pl.ds(v * 16, 16)]
            llv = ll_v[pl.ds(v * 16, 16)]
            return acc + jnp.where(va, llv + av * jnp.abs(lcv), 0.0)
        accc = lax.fori_loop(0, _NV, csum, jnp.zeros((16,), jnp.float32), unroll=13)
        row = jnp.sum(accc)
        posv = jnp.full((16,), pos.astype(jnp.float32))
        res_v[...] = jnp.where(lanes == 0, row, 0.0) / posv
        pltpu.sync_copy(res_v, out_hbm.at[wid])


def kernel(pred_bboxes, default_bboxes, gt_bboxes, a=1):
    dcx = default_bboxes[:, 0]
    dcy = default_bboxes[:, 1]
    dw = default_bboxes[:, 2]
    dh = default_bboxes[:, 3]
    dfx = jnp.stack([
        dcx - dw * 0.5, dcx + dw * 0.5, dcy - dh * 0.5, dcy + dh * 0.5,
        dw * dh, 1.0 / dw, 1.0 / dh, dcx / dw, dcy / dh,
        jnp.log(dw), jnp.log(dh)], axis=0)

    gcx = gt_bboxes[..., 0]
    gcy = gt_bboxes[..., 1]
    gw = gt_bboxes[..., 2]
    gh = gt_bboxes[..., 3]
    gtx = jnp.concatenate([
        jnp.stack([gcx, gcy, gcx - gw * 0.5, gcx + gw * 0.5,
                   gcy - gh * 0.5, gcy + gh * 0.5, gw * gh,
                   jnp.log(gw), jnp.log(gh)], axis=-1),
        gt_bboxes[..., 4:25],
        jnp.zeros((_N, _G, 2), jnp.float32)], axis=-1)

    n_tiles = (_P + _P_TILE - 1) // _P_TILE
    out2 = jax.ShapeDtypeStruct((_N, 1, _P_PAD), jnp.float32)
    lconf, lloc, many = pl.pallas_call(
        _stage1_body,
        grid=(_N, n_tiles),
        in_specs=[
            pl.BlockSpec((1, _P_TILE, 25), lambda n, t: (n, t, 0)),
            pl.BlockSpec((1, _G, 32), lambda n, t: (n, 0, 0)),
            pl.BlockSpec((11, _P_TILE), lambda n, t: (0, t)),
        ],
        out_specs=[
            pl.BlockSpec((1, 1, _P_TILE), lambda n, t: (n, 0, t)),
            pl.BlockSpec((1, 1, _P_TILE), lambda n, t: (n, 0, t)),
            pl.BlockSpec((1, 1, _P_TILE), lambda n, t: (n, 0, t)),
        ],
        out_shape=[out2, out2, out2],
        scratch_shapes=[pltpu.VMEM((_G, _P_TILE), jnp.float32)],
    )(pred_bboxes, gtx, dfx)

    a_arr = jnp.full((16,), a, jnp.float32)
    import functools
    sc_call = functools.partial(
        pl.kernel,
        out_type=jax.ShapeDtypeStruct((_N, 16), jnp.float32),
        mesh=plsc.VectorSubcoreMesh(core_axis_name="c", subcore_axis_name="s"),
        compiler_params=pltpu.CompilerParams(needs_layout_passes=False),
        scratch_types=[
            pltpu.VMEM((_P_PAD,), jnp.float32),
            pltpu.VMEM((_P_PAD,), jnp.float32),
            pltpu.VMEM((_P_PAD,), jnp.float32),
            pltpu.VMEM((_P_PAD,), jnp.int32),
            pltpu.VMEM((16,), jnp.float32),
            pltpu.VMEM((256,), jnp.int32),
            pltpu.VMEM((16,), jnp.float32),
        ],
    )(_sc_stage2_body)
    out = sc_call(lconf.reshape(_N, _P_PAD), lloc.reshape(_N, _P_PAD),
                  many.reshape(_N, _P_PAD), a_arr)
    return jnp.sum(out) / float(_N)


# speedup vs baseline: 1.2743x; 1.2108x over previous
import jax
import jax.numpy as jnp
from jax import lax
from jax.experimental import pallas as pl
from jax.experimental.pallas import tpu as pltpu
from jax.experimental.pallas import tpu_sc as plsc

_N, _P, _G, _C = 8, 8732, 64, 21
_P_TILE = 1792
_INT_MIN = -(2 ** 31)


def _stage1_body(pred_ref, gtx_ref, dfx_ref, lconf_ref, lloc_ref, many_ref,
                 mf_ref):
    predT = pred_ref[0].T
    logits = predT[4:25, :]
    m = jnp.max(logits, axis=0, keepdims=True)
    lse = jnp.log(jnp.sum(jnp.exp(logits - m), axis=0, keepdims=True)) + m
    logp = logits - lse

    gtx = gtx_ref[0]
    gclsT = gtx[:, 9:30].T

    d_l = dfx_ref[0:1, :]
    d_r = dfx_ref[1:2, :]
    d_b = dfx_ref[2:3, :]
    d_t = dfx_ref[3:4, :]
    d_area = dfx_ref[4:5, :]
    inv_dw = dfx_ref[5:6, :]
    inv_dh = dfx_ref[6:7, :]

    a_cx = predT[0:1, :] + dfx_ref[7:8, :]
    a_cy = predT[1:2, :] + dfx_ref[8:9, :]
    a_w = predT[2:3, :] + dfx_ref[9:10, :]
    a_h = predT[3:4, :] + dfx_ref[10:11, :]

    def sl1(x):
        ax = jnp.abs(x)
        mm = jnp.minimum(ax, 1.0)
        return (ax - mm) + (0.5 * mm) * mm

    acc = jnp.zeros((8, _P_TILE), jnp.float32)
    for c in range(_G // 8):
        sl = slice(c * 8, c * 8 + 8)
        g_cx = gtx[sl, 0:1]
        g_cy = gtx[sl, 1:2]
        g_l = gtx[sl, 2:3]
        g_r = gtx[sl, 3:4]
        g_b = gtx[sl, 4:5]
        g_t = gtx[sl, 5:6]
        g_area = gtx[sl, 6:7]
        log_gw = gtx[sl, 7:8]
        log_gh = gtx[sl, 8:9]
        w = jnp.maximum(jnp.minimum(g_r, d_r) - jnp.maximum(g_l, d_l), 0.0)
        h = jnp.maximum(jnp.minimum(g_t, d_t) - jnp.maximum(g_b, d_b), 0.0)
        inter = w * h
        match = (3.0 * inter) > (g_area + d_area)
        mf_ref[sl, :] = match.astype(jnp.float32)
        x1 = a_cx - g_cx * inv_dw
        x2 = a_cy - g_cy * inv_dh
        x3 = a_w - log_gw
        x4 = a_h - log_gh
        s = sl1(x1) + sl1(x2) + sl1(x3) + sl1(x4)
        acc = acc + jnp.where(match, s, 0.0)

    lloc = jnp.sum(acc, axis=0, keepdims=True)
    mm_ = jnp.dot(gclsT, mf_ref[...], preferred_element_type=jnp.float32)
    cnt = jnp.sum(mm_, axis=0, keepdims=True)
    lcp = jnp.sum(logp * mm_, axis=0, keepdims=True)
    lc = jnp.where(cnt > 0.0, -lcp, logp[0:1, :])

    lconf_ref[0] = lc
    lloc_ref[0] = lloc
    many_ref[0] = (cnt > 0.0).astype(jnp.float32)


_P_PAD = 8736
_NV = _P_PAD // 16


def _sc_stage2_body(lconf_hbm, lloc_hbm, many_hbm, a_hbm, out_hbm,
                    lc_v, ll_v, my_v, uk_v, av_v, hist_v, res_v):
    imin = jnp.int32(_INT_MIN)
    wid = lax.axis_index("s") * 2 + lax.axis_index("c")

    @pl.when(wid < _N)
    def _work():
        lanes = lax.iota(jnp.int32, 16)
        pltpu.sync_copy(lconf_hbm.at[wid], lc_v)
        pltpu.sync_copy(lloc_hbm.at[wid], ll_v)
        pltpu.sync_copy(many_hbm.at[wid], my_v)
        pltpu.sync_copy(a_hbm, av_v)

        def build(v, _):
            f = lax.bitcast_convert_type(lc_v[pl.ds(v * 16, 16)], jnp.int32)
            u = jnp.where(f < 0, ~f, f ^ imin)
            u = jnp.where(v * 16 + lanes < _P, u, jnp.int32(-1))
            uk_v[pl.ds(v * 16, 16)] = u
            return 0
        lax.fori_loop(0, _NV, build, 0, unroll=13)

        def cntm(v, acc):
            mv = my_v[pl.ds(v * 16, 16)]
            return acc + jnp.where(v * 16 + lanes < _P, mv, 0.0)
        posf = jnp.sum(lax.fori_loop(0, _NV, cntm, jnp.zeros((16,), jnp.float32), unroll=13))
        pos_orig = posf.astype(jnp.int32)
        pos = jnp.maximum(pos_orig, 1)
        neg = jnp.maximum(jnp.minimum(_P - pos_orig, 3 * pos), 1)

        ones_i = jnp.ones((16,), jnp.int32)

        def select2(k_lo0, k_hi0):
            def refine(k, hoff):
                bincnt = jnp.zeros((16,), jnp.int32)
                for b in range(16):
                    cb = jnp.sum(hist_v[pl.ds(hoff + b * 16, 16)])
                    bincnt = jnp.where(lanes == b, cb, bincnt)
                incl = plsc.cumsum(bincnt)
                excl = incl - bincnt
                bsel = plsc.all_reduce_ffs(incl >= k)
                bs = bsel if getattr(bsel, "ndim", 0) == 0 else jnp.max(bsel)
                below = jnp.sum(jnp.where(lanes == bs, excl, 0))
                return bs, below

            def pass_body(p, carry):
                p_lo, k_lo, p_hi, k_hi, hmask = carry
                shift = 28 - p * 4
                for b in range(32):
                    hist_v[pl.ds(b * 16, 16)] = jnp.zeros((16,), jnp.int32)

                def scan(v, _):
                    u = uk_v[pl.ds(v * 16, 16)]
                    shv = jnp.full((16,), shift, jnp.int32)
                    idx = (lax.shift_right_logical(u, shv) & 15) * 16 + lanes
                    masked = u & hmask
                    plsc.addupdate_scatter(hist_v, [idx], ones_i,
                                           mask=masked == p_lo)
                    plsc.addupdate_scatter(hist_v, [idx + 256], ones_i,
                                           mask=masked == p_hi)
                    return 0
                lax.fori_loop(0, _NV, scan, 0, unroll=13)

                bs_lo, below_lo = refine(k_lo, 0)
                bs_hi, below_hi = refine(k_hi, 256)
                p_lo = p_lo | lax.shift_left(bs_lo, shift)
                p_hi = p_hi | lax.shift_left(bs_hi, shift)
                hmask = hmask | lax.shift_left(jnp.int32(15), shift)
                return p_lo, k_lo - below_lo, p_hi, k_hi - below_hi, hmask

            p_lo, _, p_hi, _, _ = lax.fori_loop(
                0, 8, pass_body,
                (jnp.int32(0), k_lo0, jnp.int32(0), k_hi0, jnp.int32(0)))
            return p_lo, p_hi

        uk_lo, uk_hi = select2(neg, _P - pos + 1)
        slo = uk_lo ^ imin
        shi = uk_hi ^ imin

        av = av_v[...]

        def csum(v, acc):
            sk = uk_v[pl.ds(v * 16, 16)] ^ imin
            va = ((sk < slo) | (sk > shi)) & (v * 16 + lanes < _P)
            lcv = lc_v[pl.ds(v * 16, 16)]
            llv = ll_v[pl.ds(v * 16, 16)]
            return acc + jnp.where(va, llv + av * jnp.abs(lcv), 0.0)
        accc = lax.fori_loop(0, _NV, csum, jnp.zeros((16,), jnp.float32), unroll=13)
        row = jnp.sum(accc)
        posv = jnp.full((16,), pos.astype(jnp.float32))
        res_v[...] = jnp.where(lanes == 0, row, 0.0) / posv
        pltpu.sync_copy(res_v, out_hbm.at[wid])


def kernel(pred_bboxes, default_bboxes, gt_bboxes, a=1):
    dcx = default_bboxes[:, 0]
    dcy = default_bboxes[:, 1]
    dw = default_bboxes[:, 2]
    dh = default_bboxes[:, 3]
    dfx = jnp.stack([
        dcx - dw * 0.5, dcx + dw * 0.5, dcy - dh * 0.5, dcy + dh * 0.5,
        dw * dh, 1.0 / dw, 1.0 / dh, dcx / dw, dcy / dh,
        jnp.log(dw), jnp.log(dh)], axis=0)

    gcx = gt_bboxes[..., 0]
    gcy = gt_bboxes[..., 1]
    gw = gt_bboxes[..., 2]
    gh = gt_bboxes[..., 3]
    gtx = jnp.concatenate([
        jnp.stack([gcx, gcy, gcx - gw * 0.5, gcx + gw * 0.5,
                   gcy - gh * 0.5, gcy + gh * 0.5, gw * gh,
                   jnp.log(gw), jnp.log(gh)], axis=-1),
        gt_bboxes[..., 4:25],
        jnp.zeros((_N, _G, 2), jnp.float32)], axis=-1)

    n_tiles = (_P + _P_TILE - 1) // _P_TILE
    out2 = jax.ShapeDtypeStruct((_N, 1, _P_PAD), jnp.float32)
    lconf, lloc, many = pl.pallas_call(
        _stage1_body,
        grid=(_N, n_tiles),
        in_specs=[
            pl.BlockSpec((1, _P_TILE, 25), lambda n, t: (n, t, 0)),
            pl.BlockSpec((1, _G, 32), lambda n, t: (n, 0, 0)),
            pl.BlockSpec((11, _P_TILE), lambda n, t: (0, t)),
        ],
        out_specs=[
            pl.BlockSpec((1, 1, _P_TILE), lambda n, t: (n, 0, t)),
            pl.BlockSpec((1, 1, _P_TILE), lambda n, t: (n, 0, t)),
            pl.BlockSpec((1, 1, _P_TILE), lambda n, t: (n, 0, t)),
        ],
        out_shape=[out2, out2, out2],
        scratch_shapes=[pltpu.VMEM((_G, _P_TILE), jnp.float32)],
    )(pred_bboxes, gtx, dfx)

    a_arr = jnp.full((16,), a, jnp.float32)
    import functools
    sc_call = functools.partial(
        pl.kernel,
        out_type=jax.ShapeDtypeStruct((_N, 16), jnp.float32),
        mesh=plsc.VectorSubcoreMesh(core_axis_name="c", subcore_axis_name="s"),
        compiler_params=pltpu.CompilerParams(needs_layout_passes=False),
        scratch_types=[
            pltpu.VMEM((_P_PAD,), jnp.float32),
            pltpu.VMEM((_P_PAD,), jnp.float32),
            pltpu.VMEM((_P_PAD,), jnp.float32),
            pltpu.VMEM((_P_PAD,), jnp.int32),
            pltpu.VMEM((16,), jnp.float32),
            pltpu.VMEM((512,), jnp.int32),
            pltpu.VMEM((16,), jnp.float32),
        ],
    )(_sc_stage2_body)
    out = sc_call(lconf.reshape(_N, _P_PAD), lloc.reshape(_N, _P_PAD),
                  many.reshape(_N, _P_PAD), a_arr)
    return jnp.sum(out) / float(_N)
